# baseline (speedup 1.0000x reference)
import dataclasses
import functools

import jax
import jax.numpy as jnp
from jax import lax
from jax.experimental import pallas as pl
from jax.experimental.pallas import tpu as pltpu
from jax.experimental.pallas import tpu_sc as plsc

_N = 10000
_NPAD = 10240
_E = 320000
_D = 128
_G = 64
_NCLS = 10

_NC = 2
_NS = 16
_NW = _NC * _NS
_FPW = _D // _NW
_EW = _E // _NW
_CHUNK = 2000
_VPC = _CHUNK // 16

_CB = 1024
_GRID = _NPAD // _CB

_SC_MESH = plsc.VectorSubcoreMesh(core_axis_name="c", subcore_axis_name="s")

_SC_CP = pltpu.CompilerParams()
if "needs_layout_passes" in pltpu.CompilerParams.__dataclass_fields__:
    _SC_CP = dataclasses.replace(_SC_CP, needs_layout_passes=False)


def _retire(dv, do_pass):
    def cond(rem):
        return jnp.any(rem)

    def body(rem):
        _, last = plsc.scan_count(dv, mask=rem)
        do_pass(last)
        return rem & jnp.logical_not(last)

    lax.while_loop(cond, body, jnp.ones((16,), jnp.bool_))


def _deg_body(dst_hbm, out_hbm, idx_v, deg_v, ones16):
    c = lax.axis_index("c")
    s = lax.axis_index("s")
    w = s * _NC + c

    def _zero(i, carry):
        deg_v[i // (_CB // 16), pl.ds((i % (_CB // 16)) * 16, 16)] = (
            jnp.zeros((16,), jnp.float32))
        return carry

    lax.fori_loop(0, _GRID * (_CB // 16), _zero, 0)

    ones16[0, :] = jnp.full((16,), 1.0, jnp.float32)
    ebase = w * _EW

    def _chunk(k, carry):
        pltpu.sync_copy(dst_hbm.at[pl.ds(ebase + k * _CHUNK, _CHUNK)], idx_v)

        def _vec(j, c2):
            dv = idx_v[pl.ds(j * 16, 16)]
            r = lax.shift_right_logical(dv, 10)
            q = lax.bitwise_and(dv, jnp.full((16,), _CB - 1, jnp.int32))

            def do_pass(mask):
                plsc.addupdate_scatter(deg_v, [r, q], ones16[0, :], mask=mask)

            _retire(dv, do_pass)
            return c2

        lax.fori_loop(0, _VPC, _vec, 0)
        return carry

    lax.fori_loop(0, _EW // _CHUNK, _chunk, 0)

    pltpu.sync_copy(deg_v, out_hbm.at[w])


_deg_kernel = functools.partial(
    pl.kernel,
    out_type=jax.ShapeDtypeStruct((_NW, _GRID, _CB), jnp.float32),
    mesh=_SC_MESH,
    compiler_params=_SC_CP,
    scratch_types=[
        pltpu.VMEM((_CHUNK,), jnp.int32),
        pltpu.VMEM((_GRID, _CB), jnp.float32),
        pltpu.VMEM((1, 16), jnp.float32),
    ],
)(_deg_body)


def _agg_body(y_hbm, src_hbm, dst_hbm, out_hbm, sv_v, dv_v, y_v, a_v):
    c = lax.axis_index("c")
    s = lax.axis_index("s")
    w = s * _NC + c

    def _zero(i, carry):
        a_v[i // (_NPAD // 16), pl.ds((i % (_NPAD // 16)) * 16, 16)] = (
            jnp.zeros((16,), jnp.float32))
        return carry

    lax.fori_loop(0, _FPW * (_NPAD // 16), _zero, 0)

    pltpu.sync_copy(y_hbm.at[w], y_v)

    def _chunk(k, carry):
        pltpu.sync_copy(src_hbm.at[pl.ds(k * _CHUNK, _CHUNK)], sv_v)
        pltpu.sync_copy(dst_hbm.at[pl.ds(k * _CHUNK, _CHUNK)], dv_v)

        def _vec(j, c2):
            sv = sv_v[pl.ds(j * 16, 16)]
            dv = dv_v[pl.ds(j * 16, 16)]

            def do_pass(mask):
                for f in range(_FPW):
                    fidx = jnp.full((16,), f, jnp.int32)
                    vals = plsc.load_gather(y_v, [fidx, sv], mask=mask)
                    plsc.addupdate_scatter(a_v, [fidx, dv], vals, mask=mask)

            _retire(dv, do_pass)
            return c2

        lax.fori_loop(0, _VPC, _vec, 0)
        return carry

    lax.fori_loop(0, _E // _CHUNK, _chunk, 0)

    pltpu.sync_copy(a_v, out_hbm.at[w])


_agg_kernel = functools.partial(
    pl.kernel,
    out_type=jax.ShapeDtypeStruct((_NW, _FPW, _NPAD), jnp.float32),
    mesh=_SC_MESH,
    compiler_params=_SC_CP,
    scratch_types=[
        pltpu.VMEM((_CHUNK,), jnp.int32),
        pltpu.VMEM((_CHUNK,), jnp.int32),
        pltpu.VMEM((_FPW, _NPAD), jnp.float32),
        pltpu.VMEM((_FPW, _NPAD), jnp.float32),
    ],
)(_agg_body)


def _dinv_body(parts_ref, dinv_ref):
    deg = 1.0 + jnp.sum(parts_ref[...], axis=0)
    dinv_ref[...] = lax.rsqrt(deg)


def _tc_dinv(degparts):
    return pl.pallas_call(
        _dinv_body,
        grid=(1,),
        in_specs=[pl.BlockSpec((_NW, _GRID, _CB), lambda j: (0, 0, 0))],
        out_specs=pl.BlockSpec((_GRID, _CB), lambda j: (0, 0)),
        out_shape=jax.ShapeDtypeStruct((_GRID, _CB), jnp.float32),
    )(degparts)


def _y1_body(x_ref, w_ref, d_ref, y_ref):
    yt = lax.dot_general(w_ref[...], x_ref[...], (((0,), (1,)), ((), ())),
                         preferred_element_type=jnp.float32)
    y_ref[...] = yt * d_ref[pl.ds(pl.program_id(0), 1), :]


def _tc_y1(xp, W1, dinv8):
    return pl.pallas_call(
        _y1_body,
        grid=(_GRID,),
        in_specs=[
            pl.BlockSpec((_CB, _D), lambda j: (j, 0)),
            pl.BlockSpec((_D, _D), lambda j: (0, 0)),
            pl.BlockSpec((_GRID, _CB), lambda j: (0, 0)),
        ],
        out_specs=pl.BlockSpec((_D, _CB), lambda j: (0, j)),
        out_shape=jax.ShapeDtypeStruct((_D, _NPAD), jnp.float32),
    )(xp, W1, dinv8)


def _h_body(a_ref, y_ref, w_ref, b_ref, d_ref, y2_ref):
    dinv = d_ref[pl.ds(pl.program_id(0), 1), :]
    h = jnp.maximum(dinv * (a_ref[...] + y_ref[...]) + b_ref[...], 0.0)
    y2_ref[...] = lax.dot_general(w_ref[...], h, (((0,), (0,)), ((), ())),
                                  preferred_element_type=jnp.float32) * dinv


def _tc_h(aggT, yT, W2, b1c, dinv8):
    return pl.pallas_call(
        _h_body,
        grid=(_GRID,),
        in_specs=[
            pl.BlockSpec((_D, _CB), lambda j: (0, j)),
            pl.BlockSpec((_D, _CB), lambda j: (0, j)),
            pl.BlockSpec((_D, _D), lambda j: (0, 0)),
            pl.BlockSpec((_D, 1), lambda j: (0, 0)),
            pl.BlockSpec((_GRID, _CB), lambda j: (0, 0)),
        ],
        out_specs=pl.BlockSpec((_D, _CB), lambda j: (0, j)),
        out_shape=jax.ShapeDtypeStruct((_D, _NPAD), jnp.float32),
    )(aggT, yT, W2, b1c, dinv8)


def _final_body(a_ref, y_ref, b_ref, d_ref, bat_ref, wc_ref, bc_ref,
                out_ref, p_acc, c_acc):
    j = pl.program_id(0)

    @pl.when(j == 0)
    def _init():
        p_acc[...] = jnp.zeros_like(p_acc)
        c_acc[...] = jnp.zeros_like(c_acc)

    dinv = d_ref[pl.ds(j, 1), :]
    h = jnp.maximum(dinv * (a_ref[...] + y_ref[...]) + b_ref[...], 0.0)
    gids = lax.broadcasted_iota(jnp.int32, (_G, 1), 0)
    onehot_t = (bat_ref[0] == gids).astype(jnp.float32)
    p_acc[...] += lax.dot_general(h, onehot_t, (((1,), (1,)), ((), ())),
                                  preferred_element_type=jnp.float32)
    ones_row = jnp.ones((1, _CB), jnp.float32)
    c_acc[...] += lax.dot_general(ones_row, onehot_t, (((1,), (1,)), ((), ())),
                                  preferred_element_type=jnp.float32)

    @pl.when(j == _GRID - 1)
    def _fin():
        pooled = p_acc[...] / jnp.maximum(c_acc[...], 1.0)
        out_ref[...] = lax.dot_general(pooled, wc_ref[...],
                                       (((0,), (0,)), ((), ())),
                                       preferred_element_type=jnp.float32
                                       ) + bc_ref[...]


def _tc_final(aggT, y2T, b2c, dinv8, bat3, Wc, bcr):
    return pl.pallas_call(
        _final_body,
        grid=(_GRID,),
        in_specs=[
            pl.BlockSpec((_D, _CB), lambda j: (0, j)),
            pl.BlockSpec((_D, _CB), lambda j: (0, j)),
            pl.BlockSpec((_D, 1), lambda j: (0, 0)),
            pl.BlockSpec((_GRID, _CB), lambda j: (0, 0)),
            pl.BlockSpec((1, 1, _CB), lambda j: (j, 0, 0)),
            pl.BlockSpec((_D, _NCLS), lambda j: (0, 0)),
            pl.BlockSpec((1, _NCLS), lambda j: (0, 0)),
        ],
        out_specs=pl.BlockSpec((_G, _NCLS), lambda j: (0, 0)),
        out_shape=jax.ShapeDtypeStruct((_G, _NCLS), jnp.float32),
        scratch_shapes=[
            pltpu.VMEM((_D, _G), jnp.float32),
            pltpu.VMEM((1, _G), jnp.float32),
        ],
    )(aggT, y2T, b2c, dinv8, bat3, Wc, bcr)


def kernel(x, edge_index, batch, W1, b1, W2, b2, Wc, bc):
    src = edge_index[0]
    dst = edge_index[1]
    xp = jnp.pad(x, ((0, _NPAD - _N), (0, 0)))
    batp = jnp.pad(batch, (0, _NPAD - _N), constant_values=_G)
    bat3 = batp.reshape(_GRID, 1, _CB)
    b1c = b1.reshape(_D, 1)
    b2c = b2.reshape(_D, 1)
    bcr = bc.reshape(1, _NCLS)

    degparts = _deg_kernel(dst)
    dinv8 = _tc_dinv(degparts)
    y1T = _tc_y1(xp, W1, dinv8)
    agg1 = _agg_kernel(y1T.reshape(_NW, _FPW, _NPAD), src, dst)
    y2T = _tc_h(agg1.reshape(_D, _NPAD), y1T, W2, b1c, dinv8)
    agg2 = _agg_kernel(y2T.reshape(_NW, _FPW, _NPAD), src, dst)
    return _tc_final(agg2.reshape(_D, _NPAD), y2T, b2c, dinv8, bat3, Wc, bcr)


# speedup vs baseline: 1.1614x; 1.1614x over previous
import dataclasses
import functools

import jax
import jax.numpy as jnp
from jax import lax
from jax.experimental import pallas as pl
from jax.experimental.pallas import tpu as pltpu
from jax.experimental.pallas import tpu_sc as plsc

_N = 10000
_NPAD = 10240
_E = 320000
_D = 128
_G = 64
_NCLS = 10

_NC = 2
_NS = 16
_NW = _NC * _NS
_FPW = _D // _NW
_EW = _E // _NW
_CHUNK = 2000
_VPC = _CHUNK // 16

_CB = 1024
_GRID = _NPAD // _CB

_SC_MESH = plsc.VectorSubcoreMesh(core_axis_name="c", subcore_axis_name="s")

_SC_CP = pltpu.CompilerParams()
if "needs_layout_passes" in pltpu.CompilerParams.__dataclass_fields__:
    _SC_CP = dataclasses.replace(_SC_CP, needs_layout_passes=False)


def _retire(dv, do_pass):
    _, last = plsc.scan_count(dv)
    do_pass(last)

    def cond(rem):
        return jnp.any(rem)

    def body(rem):
        _, nxt = plsc.scan_count(dv, mask=rem)
        do_pass(nxt)
        return rem & jnp.logical_not(nxt)

    lax.while_loop(cond, body, jnp.logical_not(last))


def _deg_body(dst_hbm, out_hbm, idx_v, deg_v, ones16):
    c = lax.axis_index("c")
    s = lax.axis_index("s")
    w = s * _NC + c

    def _zero(i, carry):
        deg_v[i // (_CB // 16), pl.ds((i % (_CB // 16)) * 16, 16)] = (
            jnp.zeros((16,), jnp.float32))
        return carry

    lax.fori_loop(0, _GRID * (_CB // 16), _zero, 0)

    ones16[0, :] = jnp.full((16,), 1.0, jnp.float32)
    ebase = w * _EW

    def _chunk(k, carry):
        pltpu.sync_copy(dst_hbm.at[pl.ds(ebase + k * _CHUNK, _CHUNK)], idx_v)

        def _vec(j, c2):
            dv = idx_v[pl.ds(j * 16, 16)]
            r = lax.shift_right_logical(dv, 10)
            q = lax.bitwise_and(dv, jnp.full((16,), _CB - 1, jnp.int32))

            def do_pass(mask):
                plsc.addupdate_scatter(deg_v, [r, q], ones16[0, :], mask=mask)

            _retire(dv, do_pass)
            return c2

        lax.fori_loop(0, _VPC, _vec, 0)
        return carry

    lax.fori_loop(0, _EW // _CHUNK, _chunk, 0)

    pltpu.sync_copy(deg_v, out_hbm.at[w])


_deg_kernel = functools.partial(
    pl.kernel,
    out_type=jax.ShapeDtypeStruct((_NW, _GRID, _CB), jnp.float32),
    mesh=_SC_MESH,
    compiler_params=_SC_CP,
    scratch_types=[
        pltpu.VMEM((_CHUNK,), jnp.int32),
        pltpu.VMEM((_GRID, _CB), jnp.float32),
        pltpu.VMEM((1, 16), jnp.float32),
    ],
)(_deg_body)


def _agg_body(y_hbm, src_hbm, dst_hbm, out_hbm, sv_v, dv_v, y_v, a_v):
    c = lax.axis_index("c")
    s = lax.axis_index("s")
    w = s * _NC + c

    def _zero(i, carry):
        a_v[i // (_NPAD // 16), pl.ds((i % (_NPAD // 16)) * 16, 16)] = (
            jnp.zeros((16,), jnp.float32))
        return carry

    lax.fori_loop(0, _FPW * (_NPAD // 16), _zero, 0)

    pltpu.sync_copy(y_hbm.at[w], y_v)

    def _chunk(k, carry):
        pltpu.sync_copy(src_hbm.at[pl.ds(k * _CHUNK, _CHUNK)], sv_v)
        pltpu.sync_copy(dst_hbm.at[pl.ds(k * _CHUNK, _CHUNK)], dv_v)

        def _vec(j, c2):
            sv = sv_v[pl.ds(j * 16, 16)]
            dv = dv_v[pl.ds(j * 16, 16)]

            fidxs = [jnp.full((16,), f, jnp.int32) for f in range(_FPW)]
            vals = [plsc.load_gather(y_v, [fidxs[f], sv])
                    for f in range(_FPW)]

            def do_pass(mask):
                for f in range(_FPW):
                    plsc.addupdate_scatter(a_v, [fidxs[f], dv], vals[f],
                                           mask=mask)

            _retire(dv, do_pass)
            return c2

        lax.fori_loop(0, _VPC, _vec, 0)
        return carry

    lax.fori_loop(0, _E // _CHUNK, _chunk, 0)

    pltpu.sync_copy(a_v, out_hbm.at[w])


_agg_kernel = functools.partial(
    pl.kernel,
    out_type=jax.ShapeDtypeStruct((_NW, _FPW, _NPAD), jnp.float32),
    mesh=_SC_MESH,
    compiler_params=_SC_CP,
    scratch_types=[
        pltpu.VMEM((_CHUNK,), jnp.int32),
        pltpu.VMEM((_CHUNK,), jnp.int32),
        pltpu.VMEM((_FPW, _NPAD), jnp.float32),
        pltpu.VMEM((_FPW, _NPAD), jnp.float32),
    ],
)(_agg_body)


def _dinv_body(parts_ref, dinv_ref):
    deg = 1.0 + jnp.sum(parts_ref[...], axis=0)
    dinv_ref[...] = lax.rsqrt(deg)


def _tc_dinv(degparts):
    return pl.pallas_call(
        _dinv_body,
        grid=(1,),
        in_specs=[pl.BlockSpec((_NW, _GRID, _CB), lambda j: (0, 0, 0))],
        out_specs=pl.BlockSpec((_GRID, _CB), lambda j: (0, 0)),
        out_shape=jax.ShapeDtypeStruct((_GRID, _CB), jnp.float32),
    )(degparts)


def _y1_body(x_ref, w_ref, d_ref, y_ref):
    yt = lax.dot_general(w_ref[...], x_ref[...], (((0,), (1,)), ((), ())),
                         preferred_element_type=jnp.float32)
    y_ref[...] = yt * d_ref[pl.ds(pl.program_id(0), 1), :]


def _tc_y1(xp, W1, dinv8):
    return pl.pallas_call(
        _y1_body,
        grid=(_GRID,),
        in_specs=[
            pl.BlockSpec((_CB, _D), lambda j: (j, 0)),
            pl.BlockSpec((_D, _D), lambda j: (0, 0)),
            pl.BlockSpec((_GRID, _CB), lambda j: (0, 0)),
        ],
        out_specs=pl.BlockSpec((_D, _CB), lambda j: (0, j)),
        out_shape=jax.ShapeDtypeStruct((_D, _NPAD), jnp.float32),
    )(xp, W1, dinv8)


def _h_body(a_ref, y_ref, w_ref, b_ref, d_ref, y2_ref):
    dinv = d_ref[pl.ds(pl.program_id(0), 1), :]
    h = jnp.maximum(dinv * (a_ref[...] + y_ref[...]) + b_ref[...], 0.0)
    y2_ref[...] = lax.dot_general(w_ref[...], h, (((0,), (0,)), ((), ())),
                                  preferred_element_type=jnp.float32) * dinv


def _tc_h(aggT, yT, W2, b1c, dinv8):
    return pl.pallas_call(
        _h_body,
        grid=(_GRID,),
        in_specs=[
            pl.BlockSpec((_D, _CB), lambda j: (0, j)),
            pl.BlockSpec((_D, _CB), lambda j: (0, j)),
            pl.BlockSpec((_D, _D), lambda j: (0, 0)),
            pl.BlockSpec((_D, 1), lambda j: (0, 0)),
            pl.BlockSpec((_GRID, _CB), lambda j: (0, 0)),
        ],
        out_specs=pl.BlockSpec((_D, _CB), lambda j: (0, j)),
        out_shape=jax.ShapeDtypeStruct((_D, _NPAD), jnp.float32),
    )(aggT, yT, W2, b1c, dinv8)


def _final_body(a_ref, y_ref, b_ref, d_ref, bat_ref, wc_ref, bc_ref,
                out_ref, p_acc, c_acc):
    j = pl.program_id(0)

    @pl.when(j == 0)
    def _init():
        p_acc[...] = jnp.zeros_like(p_acc)
        c_acc[...] = jnp.zeros_like(c_acc)

    dinv = d_ref[pl.ds(j, 1), :]
    h = jnp.maximum(dinv * (a_ref[...] + y_ref[...]) + b_ref[...], 0.0)
    gids = lax.broadcasted_iota(jnp.int32, (_G, 1), 0)
    onehot_t = (bat_ref[0] == gids).astype(jnp.float32)
    p_acc[...] += lax.dot_general(h, onehot_t, (((1,), (1,)), ((), ())),
                                  preferred_element_type=jnp.float32)
    ones_row = jnp.ones((1, _CB), jnp.float32)
    c_acc[...] += lax.dot_general(ones_row, onehot_t, (((1,), (1,)), ((), ())),
                                  preferred_element_type=jnp.float32)

    @pl.when(j == _GRID - 1)
    def _fin():
        pooled = p_acc[...] / jnp.maximum(c_acc[...], 1.0)
        out_ref[...] = lax.dot_general(pooled, wc_ref[...],
                                       (((0,), (0,)), ((), ())),
                                       preferred_element_type=jnp.float32
                                       ) + bc_ref[...]


def _tc_final(aggT, y2T, b2c, dinv8, bat3, Wc, bcr):
    return pl.pallas_call(
        _final_body,
        grid=(_GRID,),
        in_specs=[
            pl.BlockSpec((_D, _CB), lambda j: (0, j)),
            pl.BlockSpec((_D, _CB), lambda j: (0, j)),
            pl.BlockSpec((_D, 1), lambda j: (0, 0)),
            pl.BlockSpec((_GRID, _CB), lambda j: (0, 0)),
            pl.BlockSpec((1, 1, _CB), lambda j: (j, 0, 0)),
            pl.BlockSpec((_D, _NCLS), lambda j: (0, 0)),
            pl.BlockSpec((1, _NCLS), lambda j: (0, 0)),
        ],
        out_specs=pl.BlockSpec((_G, _NCLS), lambda j: (0, 0)),
        out_shape=jax.ShapeDtypeStruct((_G, _NCLS), jnp.float32),
        scratch_shapes=[
            pltpu.VMEM((_D, _G), jnp.float32),
            pltpu.VMEM((1, _G), jnp.float32),
        ],
    )(aggT, y2T, b2c, dinv8, bat3, Wc, bcr)


def kernel(x, edge_index, batch, W1, b1, W2, b2, Wc, bc):
    src = edge_index[0]
    dst = edge_index[1]
    xp = jnp.pad(x, ((0, _NPAD - _N), (0, 0)))
    batp = jnp.pad(batch, (0, _NPAD - _N), constant_values=_G)
    bat3 = batp.reshape(_GRID, 1, _CB)
    b1c = b1.reshape(_D, 1)
    b2c = b2.reshape(_D, 1)
    bcr = bc.reshape(1, _NCLS)

    degparts = _deg_kernel(dst)
    dinv8 = _tc_dinv(degparts)
    y1T = _tc_y1(xp, W1, dinv8)
    agg1 = _agg_kernel(y1T.reshape(_NW, _FPW, _NPAD), src, dst)
    y2T = _tc_h(agg1.reshape(_D, _NPAD), y1T, W2, b1c, dinv8)
    agg2 = _agg_kernel(y2T.reshape(_NW, _FPW, _NPAD), src, dst)
    return _tc_final(agg2.reshape(_D, _NPAD), y2T, b2c, dinv8, bat3, Wc, bcr)


# speedup vs baseline: 1.1890x; 1.0237x over previous
import dataclasses
import functools

import jax
import jax.numpy as jnp
from jax import lax
from jax.experimental import pallas as pl
from jax.experimental.pallas import tpu as pltpu
from jax.experimental.pallas import tpu_sc as plsc

_N = 10000
_NPAD = 10240
_E = 320000
_D = 128
_G = 64
_NCLS = 10

_NC = 2
_NS = 16
_NW = _NC * _NS
_FPW = _D // _NW
_EW = _E // _NW
_CHUNK = 2000
_VPC = _CHUNK // 16

_CB = 1024
_GRID = _NPAD // _CB

_SC_MESH = plsc.VectorSubcoreMesh(core_axis_name="c", subcore_axis_name="s")

_SC_CP = pltpu.CompilerParams()
if "needs_layout_passes" in pltpu.CompilerParams.__dataclass_fields__:
    _SC_CP = dataclasses.replace(_SC_CP, needs_layout_passes=False)


def _retire(dv, do_pass):
    _, last = plsc.scan_count(dv)
    do_pass(last)

    def cond(rem):
        return jnp.any(rem)

    def body(rem):
        _, nxt = plsc.scan_count(dv, mask=rem)
        do_pass(nxt)
        return rem & jnp.logical_not(nxt)

    lax.while_loop(cond, body, jnp.logical_not(last))


def _deg_body(dst_hbm, out_hbm, idx_v, deg_v, ones16):
    c = lax.axis_index("c")
    s = lax.axis_index("s")
    w = s * _NC + c

    def _zero(i, carry):
        deg_v[i // (_CB // 16), pl.ds((i % (_CB // 16)) * 16, 16)] = (
            jnp.zeros((16,), jnp.float32))
        return carry

    lax.fori_loop(0, _GRID * (_CB // 16), _zero, 0)

    ones16[0, :] = jnp.full((16,), 1.0, jnp.float32)
    ebase = w * _EW

    def _chunk(k, carry):
        pltpu.sync_copy(dst_hbm.at[pl.ds(ebase + k * _CHUNK, _CHUNK)], idx_v)

        def _vec(j, c2):
            dv = idx_v[pl.ds(j * 16, 16)]
            r = lax.shift_right_logical(dv, 10)
            q = lax.bitwise_and(dv, jnp.full((16,), _CB - 1, jnp.int32))

            def do_pass(mask):
                plsc.addupdate_scatter(deg_v, [r, q], ones16[0, :], mask=mask)

            _retire(dv, do_pass)
            return c2

        lax.fori_loop(0, _VPC, _vec, 0)
        return carry

    lax.fori_loop(0, _EW // _CHUNK, _chunk, 0)

    pltpu.sync_copy(deg_v, out_hbm.at[w])


_deg_kernel = functools.partial(
    pl.kernel,
    out_type=jax.ShapeDtypeStruct((_NW, _GRID, _CB), jnp.float32),
    mesh=_SC_MESH,
    compiler_params=_SC_CP,
    scratch_types=[
        pltpu.VMEM((_CHUNK,), jnp.int32),
        pltpu.VMEM((_GRID, _CB), jnp.float32),
        pltpu.VMEM((1, 16), jnp.float32),
    ],
)(_deg_body)


def _agg_body(y_hbm, src_hbm, dst_hbm, out_hbm, sv_v, dv_v, y_v, a_v):
    c = lax.axis_index("c")
    s = lax.axis_index("s")
    w = s * _NC + c

    def _zero(i, carry):
        a_v[i // (_NPAD // 16), pl.ds((i % (_NPAD // 16)) * 16, 16)] = (
            jnp.zeros((16,), jnp.float32))
        return carry

    lax.fori_loop(0, _FPW * (_NPAD // 16), _zero, 0)

    pltpu.sync_copy(y_hbm.at[w], y_v)

    def _chunk(k, carry):
        pltpu.sync_copy(src_hbm.at[pl.ds(k * _CHUNK, _CHUNK)], sv_v)
        pltpu.sync_copy(dst_hbm.at[pl.ds(k * _CHUNK, _CHUNK)], dv_v)

        @plsc.parallel_loop(0, _VPC, unroll=2)
        def _vec(j):
            sv = sv_v[pl.ds(j * 16, 16)]
            dv = dv_v[pl.ds(j * 16, 16)]

            fidxs = [jnp.full((16,), f, jnp.int32) for f in range(_FPW)]
            vals = [plsc.load_gather(y_v, [fidxs[f], sv])
                    for f in range(_FPW)]

            def do_pass(mask):
                for f in range(_FPW):
                    plsc.addupdate_scatter(a_v, [fidxs[f], dv], vals[f],
                                           mask=mask)

            _retire(dv, do_pass)

        return carry

    lax.fori_loop(0, _E // _CHUNK, _chunk, 0)

    pltpu.sync_copy(a_v, out_hbm.at[w])


_agg_kernel = functools.partial(
    pl.kernel,
    out_type=jax.ShapeDtypeStruct((_NW, _FPW, _NPAD), jnp.float32),
    mesh=_SC_MESH,
    compiler_params=_SC_CP,
    scratch_types=[
        pltpu.VMEM((_CHUNK,), jnp.int32),
        pltpu.VMEM((_CHUNK,), jnp.int32),
        pltpu.VMEM((_FPW, _NPAD), jnp.float32),
        pltpu.VMEM((_FPW, _NPAD), jnp.float32),
    ],
)(_agg_body)


def _dinv_body(parts_ref, dinv_ref):
    deg = 1.0 + jnp.sum(parts_ref[...], axis=0)
    dinv_ref[...] = lax.rsqrt(deg)


def _tc_dinv(degparts):
    return pl.pallas_call(
        _dinv_body,
        grid=(1,),
        in_specs=[pl.BlockSpec((_NW, _GRID, _CB), lambda j: (0, 0, 0))],
        out_specs=pl.BlockSpec((_GRID, _CB), lambda j: (0, 0)),
        out_shape=jax.ShapeDtypeStruct((_GRID, _CB), jnp.float32),
    )(degparts)


def _y1_body(x_ref, w_ref, d_ref, y_ref):
    yt = lax.dot_general(w_ref[...], x_ref[...], (((0,), (1,)), ((), ())),
                         preferred_element_type=jnp.float32)
    y_ref[...] = yt * d_ref[pl.ds(pl.program_id(0), 1), :]


def _tc_y1(xp, W1, dinv8):
    return pl.pallas_call(
        _y1_body,
        grid=(_GRID,),
        in_specs=[
            pl.BlockSpec((_CB, _D), lambda j: (j, 0)),
            pl.BlockSpec((_D, _D), lambda j: (0, 0)),
            pl.BlockSpec((_GRID, _CB), lambda j: (0, 0)),
        ],
        out_specs=pl.BlockSpec((_D, _CB), lambda j: (0, j)),
        out_shape=jax.ShapeDtypeStruct((_D, _NPAD), jnp.float32),
    )(xp, W1, dinv8)


def _h_body(a_ref, y_ref, w_ref, b_ref, d_ref, y2_ref):
    dinv = d_ref[pl.ds(pl.program_id(0), 1), :]
    h = jnp.maximum(dinv * (a_ref[...] + y_ref[...]) + b_ref[...], 0.0)
    y2_ref[...] = lax.dot_general(w_ref[...], h, (((0,), (0,)), ((), ())),
                                  preferred_element_type=jnp.float32) * dinv


def _tc_h(aggT, yT, W2, b1c, dinv8):
    return pl.pallas_call(
        _h_body,
        grid=(_GRID,),
        in_specs=[
            pl.BlockSpec((_D, _CB), lambda j: (0, j)),
            pl.BlockSpec((_D, _CB), lambda j: (0, j)),
            pl.BlockSpec((_D, _D), lambda j: (0, 0)),
            pl.BlockSpec((_D, 1), lambda j: (0, 0)),
            pl.BlockSpec((_GRID, _CB), lambda j: (0, 0)),
        ],
        out_specs=pl.BlockSpec((_D, _CB), lambda j: (0, j)),
        out_shape=jax.ShapeDtypeStruct((_D, _NPAD), jnp.float32),
    )(aggT, yT, W2, b1c, dinv8)


def _final_body(a_ref, y_ref, b_ref, d_ref, bat_ref, wc_ref, bc_ref,
                out_ref, p_acc, c_acc):
    j = pl.program_id(0)

    @pl.when(j == 0)
    def _init():
        p_acc[...] = jnp.zeros_like(p_acc)
        c_acc[...] = jnp.zeros_like(c_acc)

    dinv = d_ref[pl.ds(j, 1), :]
    h = jnp.maximum(dinv * (a_ref[...] + y_ref[...]) + b_ref[...], 0.0)
    gids = lax.broadcasted_iota(jnp.int32, (_G, 1), 0)
    onehot_t = (bat_ref[0] == gids).astype(jnp.float32)
    p_acc[...] += lax.dot_general(h, onehot_t, (((1,), (1,)), ((), ())),
                                  preferred_element_type=jnp.float32)
    ones_row = jnp.ones((1, _CB), jnp.float32)
    c_acc[...] += lax.dot_general(ones_row, onehot_t, (((1,), (1,)), ((), ())),
                                  preferred_element_type=jnp.float32)

    @pl.when(j == _GRID - 1)
    def _fin():
        pooled = p_acc[...] / jnp.maximum(c_acc[...], 1.0)
        out_ref[...] = lax.dot_general(pooled, wc_ref[...],
                                       (((0,), (0,)), ((), ())),
                                       preferred_element_type=jnp.float32
                                       ) + bc_ref[...]


def _tc_final(aggT, y2T, b2c, dinv8, bat3, Wc, bcr):
    return pl.pallas_call(
        _final_body,
        grid=(_GRID,),
        in_specs=[
            pl.BlockSpec((_D, _CB), lambda j: (0, j)),
            pl.BlockSpec((_D, _CB), lambda j: (0, j)),
            pl.BlockSpec((_D, 1), lambda j: (0, 0)),
            pl.BlockSpec((_GRID, _CB), lambda j: (0, 0)),
            pl.BlockSpec((1, 1, _CB), lambda j: (j, 0, 0)),
            pl.BlockSpec((_D, _NCLS), lambda j: (0, 0)),
            pl.BlockSpec((1, _NCLS), lambda j: (0, 0)),
        ],
        out_specs=pl.BlockSpec((_G, _NCLS), lambda j: (0, 0)),
        out_shape=jax.ShapeDtypeStruct((_G, _NCLS), jnp.float32),
        scratch_shapes=[
            pltpu.VMEM((_D, _G), jnp.float32),
            pltpu.VMEM((1, _G), jnp.float32),
        ],
    )(aggT, y2T, b2c, dinv8, bat3, Wc, bcr)


def kernel(x, edge_index, batch, W1, b1, W2, b2, Wc, bc):
    src = edge_index[0]
    dst = edge_index[1]
    xp = jnp.pad(x, ((0, _NPAD - _N), (0, 0)))
    batp = jnp.pad(batch, (0, _NPAD - _N), constant_values=_G)
    bat3 = batp.reshape(_GRID, 1, _CB)
    b1c = b1.reshape(_D, 1)
    b2c = b2.reshape(_D, 1)
    bcr = bc.reshape(1, _NCLS)

    degparts = _deg_kernel(dst)
    dinv8 = _tc_dinv(degparts)
    y1T = _tc_y1(xp, W1, dinv8)
    agg1 = _agg_kernel(y1T.reshape(_NW, _FPW, _NPAD), src, dst)
    y2T = _tc_h(agg1.reshape(_D, _NPAD), y1T, W2, b1c, dinv8)
    agg2 = _agg_kernel(y2T.reshape(_NW, _FPW, _NPAD), src, dst)
    return _tc_final(agg2.reshape(_D, _NPAD), y2T, b2c, dinv8, bat3, Wc, bcr)
